# Initial kernel scaffold; baseline (speedup 1.0000x reference)
import functools

import jax
import jax.numpy as jnp
import numpy as np
from jax import lax
from jax.experimental import pallas as pl
from jax.experimental.pallas import tpu as pltpu
from jax.experimental.pallas import tpu_sc as plsc

_N = 100000
_E = _N * 32
_H = 0.05
_REST_DENSITY = 1000.0
_ALPHA = 0.01
_DELTA = 0.1
_GAMMA = 7.0
_C0 = 10.0 * float(np.sqrt(2.0 * 9.81 * 0.3))
_EPS = _H * _H * 0.1
_DX = _H * 0.5
_MASS = _REST_DENSITY * _DX * _DX
_CK = 7.0 / (4.0 * np.pi * _H * _H)

_NP = 100352
_CHUNK = 2048
_CHUNKS = 49
_TILES = 32
_EP = _TILES * _CHUNK * _CHUNKS
_ROWS_PER_TILE = _CHUNK * _CHUNKS // 128
_NSLICE = _NP // 16

_INV_H = 1.0 / _H
_NEG5CK = -5.0 * _CK
_PK = _REST_DENSITY * _C0 * _C0 / _GAMMA
_DHC0M2 = _DELTA * _H * _C0 * 2.0 * _MASS
_AHC0 = _ALPHA * _H * _C0


def _prep_body(feat_ref, tbl_ref):
    pos = feat_ref[0:2, :]
    vel = feat_ref[2:4, :]
    rho = feat_ref[4:5, :]
    x = rho * (1.0 / _REST_DENSITY)
    x2 = x * x
    x4 = x2 * x2
    press = _PK * (x4 * x2 * x - 1.0)
    inv_rho = 1.0 / rho
    tbl_ref[0:2, :] = pos
    tbl_ref[2:4, :] = vel
    tbl_ref[4:5, :] = rho
    tbl_ref[5:6, :] = press * inv_rho * inv_rho
    tbl_ref[6:7, :] = inv_rho
    tbl_ref[7:8, :] = press


def _prep(feats):
    return pl.pallas_call(
        _prep_body,
        grid=(_NP // 1024,),
        in_specs=[pl.BlockSpec((8, 1024), lambda b: (0, b))],
        out_specs=pl.BlockSpec((8, 1024), lambda b: (0, b)),
        out_shape=jax.ShapeDtypeStruct((8, _NP), jnp.float32),
    )(feats)


def _rsqrt(x):
    i = plsc.bitcast(x, jnp.int32)
    i = jnp.int32(0x5F3759DF) - lax.shift_right_logical(i, 1)
    y = plsc.bitcast(i, jnp.float32)
    for _ in range(3):
        y = y * (1.5 - 0.5 * x * y * y)
    return y


def _sc_body(tbl_hbm, idxi_hbm, idxj_hbm, zeros_hbm, part_hbm,
             idxi_v, idxj_v, rowsi_v, rowsj_v, vals_v, acc_sh, sem):
    c = lax.axis_index("c")
    s = lax.axis_index("s")
    wid = s * 2 + c

    pltpu.sync_copy(zeros_hbm.at[pl.ds(s * _NSLICE, _NSLICE)],
                    acc_sh.at[pl.ds(s * _NSLICE, _NSLICE)])
    plsc.subcore_barrier()

    iota16 = lax.iota(jnp.int32, 16)

    def chunk_body(ci, carry):
        row0 = wid * _ROWS_PER_TILE + ci * 16
        pltpu.sync_copy(idxi_hbm.at[pl.ds(row0, 16)], idxi_v)
        pltpu.sync_copy(idxj_hbm.at[pl.ds(row0, 16)], idxj_v)
        descs = []
        for k in range(16):
            descs.append(pltpu.async_copy(
                tbl_hbm.at[idxi_v.at[k]], rowsi_v.at[pl.ds(k * 128, 128)], sem))
            descs.append(pltpu.async_copy(
                tbl_hbm.at[idxj_v.at[k]], rowsj_v.at[pl.ds(k * 128, 128)], sem))
        for d in descs:
            d.wait()

        def grp(g, carry2):
            rid = g * 16 + iota16

            def col(ref, cc):
                return plsc.load_gather(ref, [rid, jnp.full((16,), cc, jnp.int32)])

            pix = col(rowsi_v, 0); piy = col(rowsi_v, 1)
            vix = col(rowsi_v, 2); viy = col(rowsi_v, 3)
            rhoi = col(rowsi_v, 4); pri = col(rowsi_v, 5)
            pjx = col(rowsj_v, 0); pjy = col(rowsj_v, 1)
            vjx = col(rowsj_v, 2); vjy = col(rowsj_v, 3)
            rhoj = col(rowsj_v, 4); prj = col(rowsj_v, 5)
            irj = col(rowsj_v, 6)

            xx = pix - pjx
            xy = piy - pjy
            r2 = xx * xx + xy * xy
            rs = r2 + 1e-12
            r = rs * _rsqrt(rs)
            q = jnp.minimum(r * _INV_H, 2.0)
            t = 1.0 - 0.5 * q
            dwdq = _NEG5CK * q * (t * t * t)
            gw = dwdq / (_H * (r + 1e-7))
            gwx = gw * xx
            gwy = gw * xy
            vx = vix - vjx
            vy = viy - vjy
            inv_r2e = 1.0 / (r2 + _EPS)
            xdotg = xx * gwx + xy * gwy
            c0v = _MASS * (vx * gwx + vy * gwy) + \
                _DHC0M2 * irj * (rhoj - rhoi) * xdotg * inv_r2e
            vdotx = vx * xx + vy * xy
            pi_ij = _AHC0 * vdotx * inv_r2e / (0.5 * (rhoi + rhoj))
            pi_ij = jnp.where(vdotx < 0.0, pi_ij, jnp.zeros((16,), jnp.float32))
            f = -_MASS * (pi_ij + pri + prj)

            plsc.store_scatter(vals_v, [rid, jnp.full((16,), 0, jnp.int32)], c0v)
            plsc.store_scatter(vals_v, [rid, jnp.full((16,), 1, jnp.int32)], f * gwx)
            plsc.store_scatter(vals_v, [rid, jnp.full((16,), 2, jnp.int32)], f * gwy)
            plsc.store_scatter(vals_v, [rid, jnp.full((16,), 3, jnp.int32)],
                               jnp.zeros((16,), jnp.float32))
            return carry2

        lax.fori_loop(0, _CHUNK // 16, grp, 0)

        for k in range(16):
            pltpu.sync_copy(vals_v.at[pl.ds(k * 128, 128)],
                            acc_sh.at[idxi_v.at[k]], add=True)
        return carry

    lax.fori_loop(0, _CHUNKS, chunk_body, 0)
    plsc.subcore_barrier()
    pltpu.sync_copy(acc_sh.at[pl.ds(s * _NSLICE, _NSLICE)],
                    part_hbm.at[c, pl.ds(s * _NSLICE, _NSLICE)])


def _sc_edge_pass(table, idx_i, idx_j, zeros):
    mesh = plsc.VectorSubcoreMesh(core_axis_name="c", subcore_axis_name="s")
    k = functools.partial(
        pl.kernel,
        out_type=jax.ShapeDtypeStruct((2, _NP, 4), jnp.float32),
        mesh=mesh,
        scratch_types=[
            pltpu.VMEM((16, 128), jnp.int32),
            pltpu.VMEM((16, 128), jnp.int32),
            pltpu.VMEM((_CHUNK, 8), jnp.float32),
            pltpu.VMEM((_CHUNK, 8), jnp.float32),
            pltpu.VMEM((_CHUNK, 4), jnp.float32),
            pltpu.VMEM_SHARED((_NP, 4), jnp.float32),
            pltpu.SemaphoreType.DMA,
        ],
    )(_sc_body)
    return k(table, idx_i, idx_j, zeros)


def _combine_body(p0_ref, p1_ref, tbl_ref, out_ref):
    ssum = p0_ref[...] + p1_ref[...]
    press = jnp.broadcast_to(tbl_ref[7:8, :], (4, 1024))
    rows = lax.broadcasted_iota(jnp.int32, (4, 1024), 0)
    out_ref[...] = jnp.where(rows < 3, ssum, press)


def _combine(p0t, p1t, tblt):
    return pl.pallas_call(
        _combine_body,
        grid=(_NP // 1024,),
        in_specs=[
            pl.BlockSpec((4, 1024), lambda b: (0, b)),
            pl.BlockSpec((4, 1024), lambda b: (0, b)),
            pl.BlockSpec((8, 1024), lambda b: (0, b)),
        ],
        out_specs=pl.BlockSpec((4, 1024), lambda b: (0, b)),
        out_shape=jax.ShapeDtypeStruct((4, _NP), jnp.float32),
    )(p0t, p1t, tblt)


def kernel(positions, velocities, densities, edge_index):
    pad_n = _NP - _N
    feats = jnp.concatenate(
        [positions.T, velocities.T, densities.reshape(1, _N)], axis=0)
    feats = jnp.pad(feats, ((0, 3), (0, pad_n)), constant_values=1.0)

    tblt = _prep(feats)
    table = tblt.T

    ii = jnp.pad(edge_index[0], (0, _EP - _E)).reshape(_EP // 128, 128)
    jj = jnp.pad(edge_index[1], (0, _EP - _E)).reshape(_EP // 128, 128)

    partials = _sc_edge_pass(table, ii, jj, jnp.zeros((_NP, 4), jnp.float32))

    outt = _combine(partials[0].T, partials[1].T, tblt)
    return outt[:, :_N].T


# speedup vs baseline: 98.4016x; 98.4016x over previous
import functools

import jax
import jax.numpy as jnp
import numpy as np
from jax import lax
from jax.experimental import pallas as pl
from jax.experimental.pallas import tpu as pltpu
from jax.experimental.pallas import tpu_sc as plsc

_N = 100000
_E = _N * 32
_H = 0.05
_REST_DENSITY = 1000.0
_ALPHA = 0.01
_DELTA = 0.1
_GAMMA = 7.0
_C0 = 10.0 * float(np.sqrt(2.0 * 9.81 * 0.3))
_EPS = _H * _H * 0.1
_DX = _H * 0.5
_MASS = _REST_DENSITY * _DX * _DX
_CK = 7.0 / (4.0 * np.pi * _H * _H)

_NP = 100352
_CHUNK = 2048
_CHUNKS = 49
_TILES = 32
_EP = _TILES * _CHUNK * _CHUNKS
_ROWS_PER_TILE = _CHUNK * _CHUNKS // 128
_NSLICE = _NP // 16

_INV_H = 1.0 / _H
_NEG5CK = -5.0 * _CK
_PK = _REST_DENSITY * _C0 * _C0 / _GAMMA
_DHC0M2 = _DELTA * _H * _C0 * 2.0 * _MASS
_AHC0 = _ALPHA * _H * _C0


def _prep_body(feat_ref, tbl_ref):
    rho = feat_ref[4:5, :]
    x = rho * (1.0 / _REST_DENSITY)
    x2 = x * x
    x4 = x2 * x2
    press = _PK * (x4 * x2 * x - 1.0)
    inv_rho = 1.0 / rho
    tbl_ref[0:4, :] = feat_ref[0:4, :]
    tbl_ref[4:5, :] = rho
    tbl_ref[5:6, :] = press * inv_rho * inv_rho
    tbl_ref[6:7, :] = inv_rho
    tbl_ref[7:8, :] = press


def _prep(feats):
    return pl.pallas_call(
        _prep_body,
        grid=(_NP // 1024,),
        in_specs=[pl.BlockSpec((8, 1024), lambda b: (0, b))],
        out_specs=pl.BlockSpec((8, 1024), lambda b: (0, b)),
        out_shape=jax.ShapeDtypeStruct((8, _NP), jnp.float32),
    )(feats)


def _rsqrt(x):
    i = plsc.bitcast(x, jnp.int32)
    i = jnp.int32(0x5F3759DF) - lax.shift_right_logical(i, 1)
    y = plsc.bitcast(i, jnp.float32)
    for _ in range(3):
        y = y * (1.5 - 0.5 * x * y * y)
    return y


def _sc_body(px_h, py_h, vx_h, vy_h, rho_h, pr_h, idxi_h, idxj_h, part_h,
             idxi_v, idxj_v, gi, gj, vb, acc, stage, sem):
    c = lax.axis_index("c")
    s = lax.axis_index("s")
    wid = s * 2 + c

    def zloop(z, carry):
        stage[pl.ds(z * 16, 16)] = jnp.zeros((16,), jnp.float32)
        return carry
    lax.fori_loop(0, _NSLICE // 16, zloop, 0)
    for a in range(3):
        pltpu.sync_copy(stage, acc[a].at[pl.ds(s * _NSLICE, _NSLICE)])
    plsc.subcore_barrier()

    cols = (px_h, py_h, vx_h, vy_h, rho_h, pr_h)

    def chunk_body(ci, carry):
        row0 = wid * _ROWS_PER_TILE + ci * 16
        pltpu.sync_copy(idxi_h.at[pl.ds(row0, 16)], idxi_v)
        pltpu.sync_copy(idxj_h.at[pl.ds(row0, 16)], idxj_v)
        descs = []
        for k in range(16):
            iri = idxi_v.at[k]
            irj = idxj_v.at[k]
            for t in range(6):
                descs.append(pltpu.async_copy(
                    cols[t].at[iri], gi[t].at[pl.ds(k * 128, 128)], sem))
                descs.append(pltpu.async_copy(
                    cols[t].at[irj], gj[t].at[pl.ds(k * 128, 128)], sem))
        for d in descs:
            d.wait()

        def grp(g, carry2):
            sl = pl.ds(g * 16, 16)
            pix = gi[0][sl]; piy = gi[1][sl]
            vix = gi[2][sl]; viy = gi[3][sl]
            rhoi = gi[4][sl]; pri = gi[5][sl]
            pjx = gj[0][sl]; pjy = gj[1][sl]
            vjx = gj[2][sl]; vjy = gj[3][sl]
            rhoj = gj[4][sl]; prj = gj[5][sl]

            xx = pix - pjx
            xy = piy - pjy
            r2 = xx * xx + xy * xy
            rs = r2 + 1e-12
            r = rs * _rsqrt(rs)
            q = jnp.minimum(r * _INV_H, 2.0)
            t1 = 1.0 - 0.5 * q
            dwdq = _NEG5CK * q * (t1 * t1 * t1)
            gw = dwdq / (_H * (r + 1e-7))
            gwx = gw * xx
            gwy = gw * xy
            vvx = vix - vjx
            vvy = viy - vjy
            inv_r2e = 1.0 / (r2 + _EPS)
            xdotg = xx * gwx + xy * gwy
            c0v = _MASS * (vvx * gwx + vvy * gwy) + \
                _DHC0M2 * (rhoj - rhoi) * xdotg * inv_r2e / rhoj
            vdotx = vvx * xx + vvy * xy
            pi_ij = _AHC0 * vdotx * inv_r2e / (0.5 * (rhoi + rhoj))
            pi_ij = jnp.where(vdotx < 0.0, pi_ij, jnp.zeros((16,), jnp.float32))
            f = -_MASS * (pi_ij + pri + prj)

            vb[0][sl] = c0v
            vb[1][sl] = f * gwx
            vb[2][sl] = f * gwy
            return carry2

        lax.fori_loop(0, _CHUNK // 16, grp, 0)

        sdescs = []
        for k in range(16):
            iri = idxi_v.at[k]
            for a in range(3):
                sdescs.append(pltpu.async_copy(
                    vb[a].at[pl.ds(k * 128, 128)], acc[a].at[iri], sem,
                    add=True))
        for d in sdescs:
            d.wait()
        return carry

    lax.fori_loop(0, _CHUNKS, chunk_body, 0)
    plsc.subcore_barrier()
    for a in range(3):
        pltpu.sync_copy(acc[a].at[pl.ds(s * _NSLICE, _NSLICE)], stage)
        pltpu.sync_copy(
            stage,
            part_h.at[pl.ds((c * 3 + a) * _NP + s * _NSLICE, _NSLICE)])


def _sc_edge_pass(cols, idx_i, idx_j):
    mesh = plsc.VectorSubcoreMesh(core_axis_name="c", subcore_axis_name="s")
    k = functools.partial(
        pl.kernel,
        out_type=jax.ShapeDtypeStruct((2 * 3 * _NP,), jnp.float32),
        mesh=mesh,
        compiler_params=pltpu.CompilerParams(needs_layout_passes=False),
        scratch_types=[
            pltpu.VMEM((16, 128), jnp.int32),
            pltpu.VMEM((16, 128), jnp.int32),
            [pltpu.VMEM((_CHUNK,), jnp.float32) for _ in range(6)],
            [pltpu.VMEM((_CHUNK,), jnp.float32) for _ in range(6)],
            [pltpu.VMEM((_CHUNK,), jnp.float32) for _ in range(3)],
            [pltpu.VMEM_SHARED((_NP,), jnp.float32) for _ in range(3)],
            pltpu.VMEM((_NSLICE,), jnp.float32),
            pltpu.SemaphoreType.DMA,
        ],
    )(_sc_body)
    return k(*cols, idx_i, idx_j)


def _combine_body(p0_ref, p1_ref, tbl_ref, out_ref):
    out_ref[0:3, :] = p0_ref[...] + p1_ref[...]
    out_ref[3:4, :] = tbl_ref[7:8, :]


def _combine(p0t, p1t, tblt):
    return pl.pallas_call(
        _combine_body,
        grid=(_NP // 1024,),
        in_specs=[
            pl.BlockSpec((3, 1024), lambda b: (0, b)),
            pl.BlockSpec((3, 1024), lambda b: (0, b)),
            pl.BlockSpec((8, 1024), lambda b: (0, b)),
        ],
        out_specs=pl.BlockSpec((4, 1024), lambda b: (0, b)),
        out_shape=jax.ShapeDtypeStruct((4, _NP), jnp.float32),
    )(p0t, p1t, tblt)


def kernel(positions, velocities, densities, edge_index):
    pad_n = _NP - _N
    feats = jnp.concatenate(
        [positions.T, velocities.T, densities.reshape(1, _N)], axis=0)
    feats = jnp.pad(feats, ((0, 3), (0, pad_n)), constant_values=1.0)

    tblt = _prep(feats)
    cols = tuple(tblt[t] for t in range(6))

    ii = jnp.pad(edge_index[0], (0, _EP - _E)).reshape(_EP // 128, 128)
    jj = jnp.pad(edge_index[1], (0, _EP - _E)).reshape(_EP // 128, 128)

    partials = _sc_edge_pass(cols, ii, jj).reshape(2, 3, _NP)

    outt = _combine(partials[0], partials[1], tblt)
    return outt[:, :_N].T
